# Initial kernel scaffold; baseline (speedup 1.0000x reference)
import jax
import jax.numpy as jnp
from jax.experimental import pallas as pl


def kernel(x, nb_x, w_edge, emb_table, edge_table, eta_table, ln_gamma, ln_beta, fc_W, fc_b):
    raise NotImplementedError("write your pallas kernel here")


# speedup vs baseline: 2.0645x; 2.0645x over previous
import functools

import jax
import jax.numpy as jnp
from jax import lax
from jax.experimental import pallas as pl
from jax.experimental.pallas import tpu as pltpu
from jax.experimental.pallas import tpu_sc as plsc

B = 1024
L = 20
NB = 16
D = 128
LP = 32
NCH = 4
CH = (L * NB) // NCH
NC = 2
NS = 16
NWORK = NC * NS
BPW = B // NWORK
DC = D // 16


def _ln_body(emb_ref, g_ref, b_ref, out_ref):
    h = emb_ref[...]
    mu = jnp.mean(h, axis=-1, keepdims=True)
    var = jnp.mean((h - mu) ** 2, axis=-1, keepdims=True)
    out_ref[...] = (h - mu) * lax.rsqrt(var + 1e-5) * g_ref[...] + b_ref[...]


def _fc_body(h_ref, w_ref, b_ref, out_ref):
    out_ref[...] = (
        jnp.dot(h_ref[...], w_ref[...], preferred_element_type=jnp.float32)
        + b_ref[...]
    )


def _sc_body(ln_tab, edge2, eta2, xp, nb2, we2, out_hbm,
             xv_all, nb_all, we_all, rows, wbuf, cen, etabuf, out_buf, sem):
    wid = lax.axis_index("s") * NC + lax.axis_index("c")
    b0 = wid * BPW

    pltpu.sync_copy(xp.at[pl.ds(b0 * LP, BPW * LP)], xv_all)
    pltpu.sync_copy(nb2.at[pl.ds(b0 * NCH, BPW * NCH)], nb_all)
    pltpu.sync_copy(we2.at[pl.ds(b0 * NCH, BPW * NCH)], we_all)

    def per_b(bl, _):
        copies = []
        for j in range(NCH):
            copies.append(pltpu.async_copy(
                ln_tab.at[nb_all.at[bl * NCH + j]],
                rows.at[pl.ds(j * CH, CH)], sem))
        for j in range(NCH):
            copies.append(pltpu.async_copy(
                edge2.at[we_all.at[bl * NCH + j]],
                wbuf.at[pl.ds(j * CH, CH)], sem))
        copies.append(pltpu.async_copy(
            ln_tab.at[xv_all.at[pl.ds(bl * LP, LP)]], cen, sem))
        copies.append(pltpu.async_copy(
            eta2.at[xv_all.at[pl.ds(bl * LP, LP)]], etabuf, sem))
        for cp in copies:
            cp.wait()

        def per_item(it, acc):
            base = it * NB
            wk = plsc.load_gather(wbuf, [jnp.full((16,), base, jnp.int32)])
            m = [wk * rows[base, pl.ds(c * 16, 16)] for c in range(DC)]
            for k in range(1, NB):
                wk = plsc.load_gather(
                    wbuf, [jnp.full((16,), base + k, jnp.int32)])
                for c in range(DC):
                    m[c] = jnp.maximum(m[c], wk * rows[base + k, pl.ds(c * 16, 16)])
            eta = plsc.load_gather(etabuf, [jnp.full((16,), it, jnp.int32)])
            om = 1.0 - eta
            return tuple(
                acc[c] + om * m[c] + eta * cen[it, pl.ds(c * 16, 16)]
                for c in range(DC))

        acc0 = tuple(jnp.zeros((16,), jnp.float32) for _ in range(DC))
        acc = lax.fori_loop(0, L, per_item, acc0)
        for c in range(DC):
            out_buf[bl, pl.ds(c * 16, 16)] = acc[c]
        return _

    lax.fori_loop(0, BPW, per_b, 0)
    pltpu.sync_copy(out_buf, out_hbm.at[pl.ds(b0, BPW)])


def kernel(x, nb_x, w_edge, emb_table, edge_table, eta_table,
           ln_gamma, ln_beta, fc_W, fc_b):
    x = x.astype(jnp.int32)
    nb_x = nb_x.astype(jnp.int32)
    w_edge = w_edge.astype(jnp.int32)

    ln_tab = pl.pallas_call(
        _ln_body,
        out_shape=jax.ShapeDtypeStruct((emb_table.shape[0], D), jnp.float32),
    )(emb_table, ln_gamma.reshape(1, D), ln_beta.reshape(1, D))

    xp = jnp.pad(x, ((0, 0), (0, LP - L))).reshape(B * LP)
    nb2 = nb_x.reshape(B * NCH, CH)
    we2 = w_edge.reshape(B * NCH, CH)

    mesh = plsc.VectorSubcoreMesh(core_axis_name="c", subcore_axis_name="s")
    sc = functools.partial(
        pl.kernel,
        mesh=mesh,
        compiler_params=pltpu.CompilerParams(needs_layout_passes=False),
        out_type=jax.ShapeDtypeStruct((B, D), jnp.float32),
        scratch_types=[
            pltpu.VMEM((BPW * LP,), jnp.int32),
            pltpu.VMEM((BPW * NCH, CH), jnp.int32),
            pltpu.VMEM((BPW * NCH, CH), jnp.int32),
            pltpu.VMEM((L * NB, D), jnp.float32),
            pltpu.VMEM((L * NB,), jnp.float32),
            pltpu.VMEM((LP, D), jnp.float32),
            pltpu.VMEM((LP,), jnp.float32),
            pltpu.VMEM((BPW, D), jnp.float32),
            pltpu.SemaphoreType.DMA,
        ],
    )(_sc_body)
    hsum = sc(ln_tab, edge_table.reshape(-1), eta_table.reshape(-1), xp, nb2, we2)

    scores = pl.pallas_call(
        _fc_body,
        out_shape=jax.ShapeDtypeStruct((B, fc_W.shape[0]), jnp.float32),
    )(hsum, fc_W.T, fc_b.reshape(1, -1))
    return scores
